# baseline (speedup 1.0000x reference)
import functools

import jax
import jax.numpy as jnp
from jax import lax
from jax.experimental import pallas as pl
from jax.experimental.pallas import tpu as pltpu
from jax.experimental.pallas import tpu_sc as plsc


def kernel(tokens, emb_table, pos_table):
    B, T = tokens.shape
    V, D = emb_table.shape
    info = plsc.get_sparse_core_info()
    NC, NS = info.num_cores, info.num_subcores
    NW = NC * NS
    N = B * T
    b_per_w = N // NW
    L = 16

    mesh = plsc.VectorSubcoreMesh(core_axis_name="c", subcore_axis_name="s")

    @functools.partial(
        pl.kernel,
        mesh=mesh,
        out_type=jax.ShapeDtypeStruct((B, T, D), jnp.float32),
        scratch_types=[
            pltpu.VMEM((B, T), jnp.int32),
            pltpu.VMEM((b_per_w, D), jnp.float32),
            pltpu.VMEM((b_per_w, D), jnp.float32),
            pltpu.SemaphoreType.DMA,
            pltpu.SemaphoreType.DMA,
        ],
    )
    def emb_kernel(tok_hbm, table_hbm, pos_hbm, out_hbm,
                   tok_v, rows_v, pos_v, sem, psem):
        wid = lax.axis_index("s") * NC + lax.axis_index("c")
        base = wid * b_per_w
        b = lax.div(base, T)
        t0 = lax.rem(base, T)
        pltpu.sync_copy(tok_hbm, tok_v)
        pos_cp = pltpu.async_copy(pos_hbm.at[pl.ds(t0, b_per_w)], pos_v, psem)

        def fire_group(g, carry):
            tokvec = tok_v[b, pl.ds(t0 + g * L, L)]
            for i in range(L):
                pltpu.async_copy(table_hbm.at[tokvec[i]],
                                 rows_v.at[g * L + i], sem)
            return carry

        lax.fori_loop(0, b_per_w // L, fire_group, 0)
        pltpu.make_async_copy(out_hbm.at[b, pl.ds(t0, b_per_w)],
                              rows_v, sem).wait()
        pos_cp.wait()

        def add_row(r, carry):
            for j in range(D // L):
                sl = pl.ds(j * L, L)
                rows_v[r, sl] = rows_v[r, sl] + pos_v[r, sl]
            return carry

        lax.fori_loop(0, b_per_w, add_row, 0)
        pltpu.sync_copy(rows_v, out_hbm.at[b, pl.ds(t0, b_per_w)])

    return emb_kernel(tokens, emb_table, pos_table)


# speedup vs baseline: 1.0017x; 1.0017x over previous
import functools

import jax
import jax.numpy as jnp
from jax import lax
from jax.experimental import pallas as pl
from jax.experimental.pallas import tpu as pltpu
from jax.experimental.pallas import tpu_sc as plsc


def kernel(tokens, emb_table, pos_table):
    B, T = tokens.shape
    V, D = emb_table.shape
    info = plsc.get_sparse_core_info()
    NC, NS = info.num_cores, info.num_subcores
    NW = NC * NS
    N = B * T
    b_per_w = N // NW
    L = 16

    mesh = plsc.VectorSubcoreMesh(core_axis_name="c", subcore_axis_name="s")

    @functools.partial(
        pl.kernel,
        mesh=mesh,
        out_type=jax.ShapeDtypeStruct((B, T, D), jnp.float32),
        scratch_types=[
            pltpu.VMEM((B, T), jnp.int32),
            pltpu.VMEM((b_per_w, D), jnp.float32),
            pltpu.VMEM((b_per_w, D), jnp.float32),
            pltpu.SemaphoreType.DMA,
            pltpu.SemaphoreType.DMA,
        ],
    )
    def emb_kernel(tok_hbm, table_hbm, pos_hbm, out_hbm,
                   tok_v, rows_v, pos_v, sem, psem):
        wid = lax.axis_index("s") * NC + lax.axis_index("c")
        base = wid * b_per_w
        b = lax.div(base, T)
        t0 = lax.rem(base, T)
        pltpu.sync_copy(tok_hbm, tok_v)
        pos_cp = pltpu.async_copy(pos_hbm.at[pl.ds(t0, b_per_w)], pos_v, psem)

        def fire_group(g, carry):
            tokvec = tok_v[b, pl.ds(t0 + g * L, L)]
            for i in range(L):
                pltpu.async_copy(table_hbm.at[pl.ds(tokvec[i], 1), :],
                                 rows_v.at[pl.ds(g * L + i, 1), :], sem)
            return carry

        lax.fori_loop(0, b_per_w // L, fire_group, 0)
        pltpu.make_async_copy(out_hbm.at[b, pl.ds(t0, b_per_w)],
                              rows_v, sem).wait()
        pos_cp.wait()

        def add_row(r, carry):
            for j in range(D // L):
                sl = pl.ds(j * L, L)
                rows_v[r, sl] = rows_v[r, sl] + pos_v[r, sl]
            return carry

        lax.fori_loop(0, b_per_w, add_row, 0)
        pltpu.sync_copy(rows_v, out_hbm.at[b, pl.ds(t0, b_per_w)])

    return emb_kernel(tokens, emb_table, pos_table)


# speedup vs baseline: 2.4694x; 2.4652x over previous
import functools

import jax
import jax.numpy as jnp
from jax import lax
from jax.experimental import pallas as pl
from jax.experimental.pallas import tpu as pltpu
from jax.experimental.pallas import tpu_sc as plsc


def kernel(tokens, emb_table, pos_table):
    B, T = tokens.shape
    V, D = emb_table.shape
    info = plsc.get_sparse_core_info()
    NC, NS = info.num_cores, info.num_subcores
    NW = NC * NS
    N = B * T
    b_per_w = N // NW
    L = 16
    W = 8

    table_t = emb_table.T
    pos_t = pos_table.T

    mesh = plsc.VectorSubcoreMesh(core_axis_name="c", subcore_axis_name="s")

    @functools.partial(
        pl.kernel,
        mesh=mesh,
        out_type=jax.ShapeDtypeStruct((B, D, T), jnp.float32),
        compiler_params=pltpu.CompilerParams(needs_layout_passes=False),
        scratch_types=[
            pltpu.VMEM((B, T), jnp.int32),
            pltpu.VMEM((W, D, 128), jnp.float32),
            pltpu.VMEM((D, b_per_w), jnp.float32),
            pltpu.VMEM((D, b_per_w), jnp.float32),
            pltpu.SemaphoreType.DMA,
            pltpu.SemaphoreType.DMA,
        ],
    )
    def emb_kernel(tok_hbm, table_hbm, pos_hbm, out_hbm,
                   tok_v, slab_v, col_v, pos_v, sem, psem):
        wid = lax.axis_index("s") * NC + lax.axis_index("c")
        base = wid * b_per_w
        b = lax.div(base, T)
        t0 = lax.rem(base, T)
        pltpu.sync_copy(tok_hbm, tok_v)
        pos_cp = pltpu.async_copy(pos_hbm.at[:, pl.ds(t0, b_per_w)],
                                  pos_v, psem)

        f_idx = [lax.iota(jnp.int32, L) + q * L for q in range(D // L)]

        def do_group(g, carry):
            tokvec = tok_v[b, pl.ds(t0 + g * L, L)]
            for h in range(L // W):
                cps = []
                for i in range(W):
                    t = tokvec[h * W + i]
                    off = pl.multiple_of(
                        lax.shift_right_logical(t, 7) * 128, 128)
                    cps.append(pltpu.async_copy(
                        table_hbm.at[:, pl.ds(off, 128)], slab_v.at[i], sem))
                for cp in cps:
                    cp.wait()
                for i in range(W):
                    t = tokvec[h * W + i]
                    c = jnp.full((L,), lax.rem(t, 128), jnp.int32)
                    slot = jnp.full((L,), i, jnp.int32)
                    dst = jnp.full((L,), g * L + h * W + i, jnp.int32)
                    for q in range(D // L):
                        vals = plsc.load_gather(slab_v, [slot, f_idx[q], c])
                        plsc.store_scatter(col_v, [f_idx[q], dst], vals)
            return carry

        lax.fori_loop(0, b_per_w // L, do_group, 0)
        pos_cp.wait()

        def add_feature(f, carry):
            for j in range(b_per_w // L):
                sl = pl.ds(j * L, L)
                col_v[f, sl] = col_v[f, sl] + pos_v[f, sl]
            return carry

        lax.fori_loop(0, D, add_feature, 0)
        pltpu.sync_copy(col_v, out_hbm.at[b, :, pl.ds(t0, b_per_w)])

    out = emb_kernel(tokens, table_t, pos_t)
    return out.transpose(0, 2, 1)


# speedup vs baseline: 3.1148x; 1.2613x over previous
import functools

import jax
import jax.numpy as jnp
from jax import lax
from jax.experimental import pallas as pl
from jax.experimental.pallas import tpu as pltpu
from jax.experimental.pallas import tpu_sc as plsc


def kernel(tokens, emb_table, pos_table):
    B, T = tokens.shape
    V, D = emb_table.shape
    info = plsc.get_sparse_core_info()
    NC, NS = info.num_cores, info.num_subcores
    NW = NC * NS
    N = B * T
    b_per_w = N // NW
    L = 16
    W = 8

    table_t = emb_table.T
    pos_t = pos_table.T

    mesh = plsc.VectorSubcoreMesh(core_axis_name="c", subcore_axis_name="s")

    @functools.partial(
        pl.kernel,
        mesh=mesh,
        out_type=jax.ShapeDtypeStruct((B, D, T), jnp.float32),
        compiler_params=pltpu.CompilerParams(needs_layout_passes=False),
        scratch_types=[
            pltpu.VMEM((B, T), jnp.int32),
            pltpu.VMEM((W, D, 128), jnp.float32),
            pltpu.VMEM((D, b_per_w), jnp.float32),
            pltpu.VMEM((D, b_per_w), jnp.float32),
            pltpu.SemaphoreType.DMA((W,)),
            pltpu.SemaphoreType.DMA,
        ],
    )
    def emb_kernel(tok_hbm, table_hbm, pos_hbm, out_hbm,
                   tok_v, slab_v, col_v, pos_v, sem, psem):
        wid = lax.axis_index("s") * NC + lax.axis_index("c")
        base = wid * b_per_w
        b = lax.div(base, T)
        t0 = lax.rem(base, T)
        pltpu.sync_copy(tok_hbm, tok_v)
        pos_cp = pltpu.async_copy(pos_hbm.at[:, pl.ds(t0, b_per_w)],
                                  pos_v, psem)

        f_idx = [lax.iota(jnp.int32, L) + q * L for q in range(D // L)]

        def fire(t, slot):
            off = pl.multiple_of(lax.shift_right_logical(t, 7) * 128, 128)
            pltpu.async_copy(table_hbm.at[:, pl.ds(off, 128)],
                             slab_v.at[slot], sem.at[slot])

        def extract(t, slot, dst):
            pltpu.make_async_copy(table_hbm.at[:, pl.ds(0, 128)],
                                  slab_v.at[slot], sem.at[slot]).wait()
            c = jnp.full((L,), lax.rem(t, 128), jnp.int32)
            sv = jnp.full((L,), slot, jnp.int32)
            dv = jnp.full((L,), dst, jnp.int32)
            for q in range(D // L):
                vals = plsc.load_gather(slab_v, [sv, f_idx[q], c])
                plsc.store_scatter(col_v, [f_idx[q], dv], vals)

        def do_group(g, carry):
            tokvec = tok_v[b, pl.ds(t0 + g * L, L)]
            gp = lax.max(g - 1, 0)
            tokvec_p = tok_v[b, pl.ds(t0 + gp * L, L)]
            for i in range(L):
                slot = i % W
                if i >= W:
                    extract(tokvec[i - W], slot, g * L + i - W)
                else:
                    @pl.when(g > 0)
                    def _():
                        extract(tokvec_p[i + W], slot, (g - 1) * L + i + W)
                fire(tokvec[i], slot)
            return carry

        lax.fori_loop(0, b_per_w // L, do_group, 0)
        tokvec_l = tok_v[b, pl.ds(t0 + b_per_w - L, L)]
        for i in range(W, L):
            extract(tokvec_l[i], i % W, b_per_w - L + i)
        pos_cp.wait()

        def add_feature(f, carry):
            for j in range(b_per_w // L):
                sl = pl.ds(j * L, L)
                col_v[f, sl] = col_v[f, sl] + pos_v[f, sl]
            return carry

        lax.fori_loop(0, D, add_feature, 0)
        pltpu.sync_copy(col_v, out_hbm.at[b, :, pl.ds(t0, b_per_w)])

    out = emb_kernel(tokens, table_t, pos_t)
    return out.transpose(0, 2, 1)
